# baseline (speedup 1.0000x reference)
import functools

import jax
import jax.numpy as jnp
from jax.experimental import pallas as pl
from jax.experimental.pallas import tpu as pltpu

N = 2048
E = 4096
D_NODE = 128
D_EDGE = 16

BN = 256
BE = 512


def _prologue_kernel(x_ref, e_ref, wn_ref, we_ref, pn_ref, pe_ref,
                     phie_ref, phiv_ref, xw_ref, ew_ref):
    phie_ref[...] = jax.lax.dot_general(
        pn_ref[...], e_ref[...], (((0,), (1,)), ((), ())),
        preferred_element_type=jnp.float32)
    phiv_ref[...] = jnp.dot(x_ref[...], pe_ref[...],
                            preferred_element_type=jnp.float32)
    xw_ref[...] = jnp.dot(x_ref[...], wn_ref[...],
                          preferred_element_type=jnp.float32)
    ew_ref[...] = jnp.dot(e_ref[...], we_ref[...],
                          preferred_element_type=jnp.float32)


def _node_kernel(inc_i_ref, inc_j_ref, lv_ref, xw_ref, phie_ref, x_ref,
                 bn_ref, out_ref):
    j = pl.program_id(1)
    a = inc_i_ref[...] * phie_ref[...]
    p = jax.lax.dot_general(a, inc_j_ref[...], (((1,), (1,)), ((), ())),
                            preferred_element_type=jnp.float32)
    p = p * lv_ref[...]
    contrib = jnp.dot(p, xw_ref[...], preferred_element_type=jnp.float32)

    @pl.when(j == 0)
    def _():
        out_ref[...] = x_ref[...] + bn_ref[...] + contrib

    @pl.when(j != 0)
    def _():
        out_ref[...] += contrib


def _edge_kernel(ci_ref, cj_ref, le_ref, ew_ref, phiv_ref, e_ref,
                 be_ref, out_ref):
    j = pl.program_id(1)
    d = cj_ref[...] * phiv_ref[...]
    p = jax.lax.dot_general(ci_ref[...], d, (((0,), (0,)), ((), ())),
                            preferred_element_type=jnp.float32)
    p = p * le_ref[...]
    contrib = jnp.dot(p, ew_ref[...], preferred_element_type=jnp.float32)

    @pl.when(j == 0)
    def _():
        out_ref[...] = e_ref[...] + be_ref[...] + contrib

    @pl.when(j != 0)
    def _():
        out_ref[...] += contrib


def kernel(x, node_laplacian, edge_laplacian, incidence, e, W_n, W_e,
           p_node, p_edge, b_n, b_e):
    f32 = jnp.float32
    bn2 = b_n.reshape(1, D_NODE)
    be2 = b_e.reshape(1, D_EDGE)

    phi_e, phi_v, xW, eW = pl.pallas_call(
        _prologue_kernel,
        out_shape=[
            jax.ShapeDtypeStruct((1, E), f32),
            jax.ShapeDtypeStruct((N, 1), f32),
            jax.ShapeDtypeStruct((N, D_NODE), f32),
            jax.ShapeDtypeStruct((E, D_EDGE), f32),
        ],
    )(x, e, W_n, W_e, p_node, p_edge)

    new_nodes = pl.pallas_call(
        _node_kernel,
        grid=(N // BN, N // BN),
        in_specs=[
            pl.BlockSpec((BN, E), lambda i, j: (i, 0)),
            pl.BlockSpec((BN, E), lambda i, j: (j, 0)),
            pl.BlockSpec((BN, BN), lambda i, j: (i, j)),
            pl.BlockSpec((BN, D_NODE), lambda i, j: (j, 0)),
            pl.BlockSpec((1, E), lambda i, j: (0, 0)),
            pl.BlockSpec((BN, D_NODE), lambda i, j: (i, 0)),
            pl.BlockSpec((1, D_NODE), lambda i, j: (0, 0)),
        ],
        out_specs=pl.BlockSpec((BN, D_NODE), lambda i, j: (i, 0)),
        out_shape=jax.ShapeDtypeStruct((N, D_NODE), f32),
        compiler_params=pltpu.CompilerParams(
            dimension_semantics=("parallel", "arbitrary")),
    )(incidence, incidence, node_laplacian, xW, phi_e, x, bn2)

    new_edges = pl.pallas_call(
        _edge_kernel,
        grid=(E // BE, E // BE),
        in_specs=[
            pl.BlockSpec((N, BE), lambda i, j: (0, i)),
            pl.BlockSpec((N, BE), lambda i, j: (0, j)),
            pl.BlockSpec((BE, BE), lambda i, j: (i, j)),
            pl.BlockSpec((BE, D_EDGE), lambda i, j: (j, 0)),
            pl.BlockSpec((N, 1), lambda i, j: (0, 0)),
            pl.BlockSpec((BE, D_EDGE), lambda i, j: (i, 0)),
            pl.BlockSpec((1, D_EDGE), lambda i, j: (0, 0)),
        ],
        out_specs=pl.BlockSpec((BE, D_EDGE), lambda i, j: (i, 0)),
        out_shape=jax.ShapeDtypeStruct((E, D_EDGE), f32),
        compiler_params=pltpu.CompilerParams(
            dimension_semantics=("parallel", "arbitrary")),
    )(incidence, incidence, edge_laplacian, eW, phi_v, e, be2)

    return new_nodes, new_edges


# speedup vs baseline: 1.4252x; 1.4252x over previous
import jax
import jax.numpy as jnp
from jax.experimental import pallas as pl
from jax.experimental.pallas import tpu as pltpu

N = 2048
E = 4096
D_NODE = 128
D_EDGE = 16

BN = 512
BE = 512

_F32 = jnp.float32
_BF16 = jnp.bfloat16


def _prologue_kernel(x_ref, e_ref, wn_ref, we_ref, pn_ref, pe_ref,
                     phie_ref, phiv_ref, xw_ref, ew_ref):
    phie_ref[...] = jax.lax.dot_general(
        pn_ref[...], e_ref[...], (((0,), (1,)), ((), ())),
        preferred_element_type=_F32).astype(_BF16)
    phiv_ref[...] = jnp.dot(x_ref[...], pe_ref[...],
                            preferred_element_type=_F32).astype(_BF16)
    xw_ref[...] = jnp.dot(x_ref[...], wn_ref[...],
                          preferred_element_type=_F32).astype(_BF16)
    ew_ref[...] = jnp.dot(e_ref[...], we_ref[...],
                          preferred_element_type=_F32).astype(_BF16)


def _node_kernel(inc_ref, lv_ref, xw_ref, phie_ref, x_ref, bn_ref, out_ref):
    i = pl.program_id(0)
    j = pl.program_id(1)
    a = inc_ref[pl.ds(i * BN, BN), :] * phie_ref[...]
    b = inc_ref[pl.ds(j * BN, BN), :]
    p = jax.lax.dot_general(a, b, (((1,), (1,)), ((), ())),
                            preferred_element_type=_F32)
    p = p * lv_ref[...]
    contrib = jnp.dot(p.astype(_BF16), xw_ref[pl.ds(j * BN, BN), :],
                      preferred_element_type=_F32)

    @pl.when(j == 0)
    def _():
        out_ref[...] = x_ref[pl.ds(i * BN, BN), :] + bn_ref[...] + contrib

    @pl.when(j != 0)
    def _():
        out_ref[...] += contrib


def _edge_kernel(inc_ref, le_ref, ew_ref, phiv_ref, e_ref, be_ref, out_ref):
    i = pl.program_id(0)
    j = pl.program_id(1)
    ci = inc_ref[:, pl.ds(i * BE, BE)]
    d = inc_ref[:, pl.ds(j * BE, BE)] * phiv_ref[...]
    p = jax.lax.dot_general(ci, d, (((0,), (0,)), ((), ())),
                            preferred_element_type=_F32)
    p = p * le_ref[...]
    contrib = jnp.dot(p.astype(_BF16), ew_ref[pl.ds(j * BE, BE), :],
                      preferred_element_type=_F32)

    @pl.when(j == 0)
    def _():
        out_ref[...] = e_ref[pl.ds(i * BE, BE), :] + be_ref[...] + contrib

    @pl.when(j != 0)
    def _():
        out_ref[...] += contrib


def kernel(x, node_laplacian, edge_laplacian, incidence, e, W_n, W_e,
           p_node, p_edge, b_n, b_e):
    bn2 = b_n.reshape(1, D_NODE)
    be2 = b_e.reshape(1, D_EDGE)
    inc_bf = incidence.astype(_BF16)

    phi_e, phi_v, xW, eW = pl.pallas_call(
        _prologue_kernel,
        out_shape=[
            jax.ShapeDtypeStruct((1, E), _BF16),
            jax.ShapeDtypeStruct((N, 1), _BF16),
            jax.ShapeDtypeStruct((N, D_NODE), _BF16),
            jax.ShapeDtypeStruct((E, D_EDGE), _BF16),
        ],
    )(x, e, W_n, W_e, p_node, p_edge)

    full = lambda i, j: (0, 0)

    new_nodes = pl.pallas_call(
        _node_kernel,
        grid=(N // BN, N // BN),
        in_specs=[
            pl.BlockSpec((N, E), full),
            pl.BlockSpec((BN, BN), lambda i, j: (i, j)),
            pl.BlockSpec((N, D_NODE), full),
            pl.BlockSpec((1, E), full),
            pl.BlockSpec((N, D_NODE), full),
            pl.BlockSpec((1, D_NODE), full),
        ],
        out_specs=pl.BlockSpec((BN, D_NODE), lambda i, j: (i, 0)),
        out_shape=jax.ShapeDtypeStruct((N, D_NODE), _F32),
        compiler_params=pltpu.CompilerParams(
            dimension_semantics=("parallel", "arbitrary")),
    )(inc_bf, node_laplacian, xW, phi_e, x, bn2)

    new_edges = pl.pallas_call(
        _edge_kernel,
        grid=(E // BE, E // BE),
        in_specs=[
            pl.BlockSpec((N, E), full),
            pl.BlockSpec((BE, BE), lambda i, j: (i, j)),
            pl.BlockSpec((E, D_EDGE), full),
            pl.BlockSpec((N, 1), full),
            pl.BlockSpec((E, D_EDGE), full),
            pl.BlockSpec((1, D_EDGE), full),
        ],
        out_specs=pl.BlockSpec((BE, D_EDGE), lambda i, j: (i, 0)),
        out_shape=jax.ShapeDtypeStruct((E, D_EDGE), _F32),
        compiler_params=pltpu.CompilerParams(
            dimension_semantics=("parallel", "arbitrary")),
    )(inc_bf, edge_laplacian, eW, phi_v, e, be2)

    return new_nodes, new_edges
